# baseline (speedup 1.0000x reference)
import functools

import jax
import jax.numpy as jnp
from jax import lax
from jax.experimental import pallas as pl
from jax.experimental.pallas import tpu as pltpu
from jax.experimental.pallas import tpu_sc as plsc

_EPS = 1e-09
_C = 10
_NW = 32


@functools.cache
def _make_sc(B):
    rows = B // _NW
    outw = rows * _C
    mesh = plsc.VectorSubcoreMesh(core_axis_name="c", subcore_axis_name="s")

    @functools.partial(
        pl.kernel,
        out_type=[
            jax.ShapeDtypeStruct((B * _C,), jnp.float32),
            jax.ShapeDtypeStruct((B * _C,), jnp.float32),
        ],
        mesh=mesh,
        compiler_params=pltpu.CompilerParams(needs_layout_passes=False),
        scratch_types=[
            pltpu.VMEM((rows,), jnp.int32),
            pltpu.VMEM((outw,), jnp.float32),
            pltpu.SemaphoreType.DMA,
        ],
    )
    def k(labels_hbm, mu_hbm, std_hbm, lab_v, buf_v, sem):
        wid = lax.axis_index("s") * 2 + lax.axis_index("c")
        rbase = wid * rows
        obase = wid * outw

        cp = pltpu.async_copy(labels_hbm.at[pl.ds(rbase, rows)], lab_v, sem)

        eps16 = jnp.full((16,), _EPS, jnp.float32)

        def fill(i, carry):
            for j in range(8):
                buf_v[pl.ds((i * 8 + j) * 16, 16)] = eps16
            return carry

        lax.fori_loop(0, outw // 128, fill, 0)
        pltpu.sync_copy(buf_v, std_hbm.at[pl.ds(obase, outw)])

        cp.wait()
        one16 = jnp.full((16,), jnp.float32(1.0 - _EPS), jnp.float32)

        def scat(i, carry):
            lab = lab_v[pl.ds(i * 16, 16)]
            r = lax.iota(jnp.int32, 16) + i * 16
            plsc.store_scatter(buf_v, [r * _C + lab], one16)
            return carry

        lax.fori_loop(0, rows // 16, scat, 0)
        pltpu.sync_copy(buf_v, mu_hbm.at[pl.ds(obase, outw)])

    return k


def kernel(labels, cuda):
    B = labels.shape[0]
    mu, std = _make_sc(B)(labels)
    return mu.reshape(1, B, _C), std.reshape(1, B, _C)
